# baseline (speedup 1.0000x reference)
import functools
import math

import jax
import jax.numpy as jnp
from jax import lax
from jax.experimental import pallas as pl
from jax.experimental.pallas import tpu as pltpu
from jax.experimental.pallas import tpu_sc as plsc

N_CLASS = 1000
BATCH = 4096
M = 0.2
S = 30.0
EPS = 0.1

_COS_M = math.cos(M)
_SIN_M = math.sin(M)
_TH = math.cos(M)
_MM = math.sin(math.pi - M) * M

BLK = 512
NBLK = BATCH // BLK

_NC = 2
_NS = 16
_NW = _NC * _NS


def _aam_per_row(vals, onehot):
    c_lab = jnp.sum(vals * onehot, axis=1, keepdims=True)
    sine = jnp.sqrt(jnp.clip(1.0 - c_lab * c_lab, 0.0, 1.0))
    phi = c_lab * _COS_M - sine * _SIN_M
    phi = jnp.where(c_lab - _TH > 0, phi, c_lab - _MM)
    delta = S * (phi - c_lab)
    t = S * vals
    t_mod = t + onehot * delta
    t_sum = jnp.sum(t, axis=1, keepdims=True)
    m = jnp.max(t_mod, axis=1, keepdims=True)
    sumexp = jnp.sum(jnp.exp(t_mod - m), axis=1, keepdims=True)
    lse = m + jnp.log(sumexp)
    rmean = (t_sum + delta) / N_CLASS
    nll = lse - S * phi
    smooth = lse - rmean
    return (1.0 - EPS) * nll + EPS * smooth


def _sample_body(x_ref, lab_ref, psum_ref):
    i = pl.program_id(0)
    x = x_ref[...]
    labs = lab_ref[0]
    cols = jax.lax.broadcasted_iota(jnp.int32, (BLK, N_CLASS), 1)
    onehot = (cols == labs).astype(x.dtype)
    per = _aam_per_row(x, onehot)

    @pl.when(i == 0)
    def _():
        psum_ref[...] = jnp.zeros_like(psum_ref)

    psum_ref[...] += jnp.sum(per, axis=0, keepdims=True)


def _center_body(sums_ref, cntcol_ref, csum_ref, npres_ref):
    sums = sums_ref[...]
    cnt = cntcol_ref[...]
    present = cnt > 0
    inv = jnp.where(present, 1.0 / jnp.where(present, cnt, 1.0), 0.0)
    centers = sums * inv
    rows = jax.lax.broadcasted_iota(jnp.int32, (N_CLASS, N_CLASS), 0)
    cols = jax.lax.broadcasted_iota(jnp.int32, (N_CLASS, N_CLASS), 1)
    diag = (rows == cols).astype(sums.dtype)
    per = _aam_per_row(centers, diag)
    per = jnp.where(present, per, 0.0)
    csum_ref[...] = jnp.sum(per, axis=0, keepdims=True)
    npres_ref[...] = jnp.sum(present.astype(jnp.float32), axis=0, keepdims=True)


_CSLC = 32
_NFULL = 31
_CLAST = N_CLASS - _NFULL * _CSLC
_RCHUNK = 2048
_RCHUNK8 = 1024
_L = 16


def _sc_scatter_body(x_hbm, lab_hbm, sums_hbm, cnt_hbm, labs_v, stage_v,
                     acc_v, stage8_v, acc8_v, cnt_v, nrows=BATCH):
    c = lax.axis_index("c")
    s = lax.axis_index("s")
    wid = s * _NC + c
    c0 = wid * _CSLC

    def zrow(i, carry):
        acc_v[i, pl.ds(0, _L)] = jnp.zeros((_L,), jnp.float32)
        acc_v[i, pl.ds(_L, _L)] = jnp.zeros((_L,), jnp.float32)
        return carry

    lax.fori_loop(0, N_CLASS, zrow, 0, unroll=8)

    pltpu.sync_copy(lab_hbm, labs_v)

    iota = lax.iota(jnp.int32, _L)

    def make_row(k):
        def row(g, carry):
            lr = g * _L
            lv = labs_v[pl.ds(k * _RCHUNK + lr, _L)]
            for j in range(_L):
                lab = lv[j]
                v0 = stage_v[lr + j, pl.ds(0, _L)]
                v1 = stage_v[lr + j, pl.ds(_L, _L)]
                plsc.addupdate(acc_v.at[lab, pl.ds(0, _L)], v0)
                plsc.addupdate(acc_v.at[lab, pl.ds(_L, _L)], v1)
            return carry

        return row

    @pl.when(wid < _NFULL)
    def _():
        for k in range(nrows // _RCHUNK):
            pltpu.sync_copy(
                x_hbm.at[pl.ds(k * _RCHUNK, _RCHUNK), pl.ds(c0, _CSLC)],
                stage_v,
            )
            lax.fori_loop(0, _RCHUNK // _L, make_row(k), 0, unroll=4)
        pltpu.sync_copy(acc_v, sums_hbm.at[:, pl.ds(c0, _CSLC)])

    @pl.when(wid == _NFULL)
    def _():
        def zcnt(i, carry):
            cnt_v[pl.ds(i * _L, _L)] = jnp.zeros((_L,), jnp.float32)
            return carry

        lax.fori_loop(0, 63, zcnt, 0, unroll=8)
        ones = jnp.ones((_L,), jnp.float32)

        def zrow8(i, carry):
            acc8_v[i, pl.ds(0, _L)] = jnp.zeros((_L,), jnp.float32)
            return carry

        lax.fori_loop(0, N_CLASS, zrow8, 0, unroll=8)

        for k in range(nrows // _RCHUNK8):
            pltpu.sync_copy(
                x_hbm.at[pl.ds(k * _RCHUNK8, _RCHUNK8), pl.ds(c0, _CLAST)],
                stage8_v,
            )

            def grp8(g, carry):
                lr = g * _L
                labs16 = labs_v[pl.ds(k * _RCHUNK8 + lr, _L)]
                rows = lr + iota
                cvecs = [(iota + col) & (_CLAST - 1) for col in range(_CLAST)]
                vals = [
                    plsc.load_gather(stage8_v, [rows, cv]) for cv in cvecs
                ]
                for cv, v in zip(cvecs, vals):
                    plsc.addupdate_scatter(acc8_v, [labs16, cv], v)
                plsc.addupdate_scatter(cnt_v, [labs16], ones)
                return carry

            lax.fori_loop(0, _RCHUNK8 // _L, grp8, 0, unroll=2)
        pltpu.sync_copy(acc8_v.at[:, pl.ds(0, _CLAST)],
                        sums_hbm.at[:, pl.ds(c0, _CLAST)])
        pltpu.sync_copy(cnt_v.at[pl.ds(0, N_CLASS)], cnt_hbm)


_sc_scatter = functools.partial(
    pl.kernel,
    mesh=plsc.VectorSubcoreMesh(core_axis_name="c", subcore_axis_name="s"),
    compiler_params=pltpu.CompilerParams(
        use_tc_tiling_on_sc=False, needs_layout_passes=False
    ),
    out_type=[
        jax.ShapeDtypeStruct((N_CLASS, N_CLASS), jnp.float32),
        jax.ShapeDtypeStruct((N_CLASS,), jnp.float32),
    ],
    scratch_types=[
        pltpu.VMEM((BATCH,), jnp.int32),
        pltpu.VMEM((_RCHUNK, _CSLC), jnp.float32),
        pltpu.VMEM((N_CLASS, _CSLC), jnp.float32),
        pltpu.VMEM((_RCHUNK8, _CLAST), jnp.float32),
        pltpu.VMEM((N_CLASS, _L), jnp.float32),
        pltpu.VMEM((1008,), jnp.float32),
    ],
)(_sc_scatter_body)


def kernel(x, label):
    sums, cnt = _sc_scatter(x, label)

    lab3 = label.reshape(NBLK, BLK, 1)
    psum = pl.pallas_call(
        _sample_body,
        grid=(NBLK,),
        in_specs=[
            pl.BlockSpec((BLK, N_CLASS), lambda i: (i, 0)),
            pl.BlockSpec((1, BLK, 1), lambda i: (i, 0, 0)),
        ],
        out_specs=pl.BlockSpec((1, 1), lambda i: (0, 0)),
        out_shape=jax.ShapeDtypeStruct((1, 1), jnp.float32),
    )(x, lab3)

    csum, npres = pl.pallas_call(
        _center_body,
        out_shape=[
            jax.ShapeDtypeStruct((1, 1), jnp.float32),
            jax.ShapeDtypeStruct((1, 1), jnp.float32),
        ],
    )(sums, cnt.reshape(N_CLASS, 1))

    loss = 0.5 * psum[0, 0] / BATCH + 0.5 * csum[0, 0] / npres[0, 0]
    return loss


# speedup vs baseline: 1.0447x; 1.0447x over previous
import functools
import math

import jax
import jax.numpy as jnp
from jax import lax
from jax.experimental import pallas as pl
from jax.experimental.pallas import tpu as pltpu
from jax.experimental.pallas import tpu_sc as plsc

N_CLASS = 1000
BATCH = 4096
M = 0.2
S = 30.0
EPS = 0.1

_COS_M = math.cos(M)
_SIN_M = math.sin(M)
_TH = math.cos(M)
_MM = math.sin(math.pi - M) * M

BLK = 512
NBLK = BATCH // BLK

_NC = 2
_NS = 16
_NW = _NC * _NS


def _aam_per_row(vals, onehot):
    c_lab = jnp.sum(vals * onehot, axis=1, keepdims=True)
    sine = jnp.sqrt(jnp.clip(1.0 - c_lab * c_lab, 0.0, 1.0))
    phi = c_lab * _COS_M - sine * _SIN_M
    phi = jnp.where(c_lab - _TH > 0, phi, c_lab - _MM)
    delta = S * (phi - c_lab)
    t = S * vals
    t_mod = t + onehot * delta
    t_sum = jnp.sum(t, axis=1, keepdims=True)
    m = jnp.max(t_mod, axis=1, keepdims=True)
    sumexp = jnp.sum(jnp.exp(t_mod - m), axis=1, keepdims=True)
    lse = m + jnp.log(sumexp)
    rmean = (t_sum + delta) / N_CLASS
    nll = lse - S * phi
    smooth = lse - rmean
    return (1.0 - EPS) * nll + EPS * smooth


def _sample_body(x_ref, lab_ref, psum_ref):
    i = pl.program_id(0)
    x = x_ref[...]
    labs = lab_ref[0]
    cols = jax.lax.broadcasted_iota(jnp.int32, (BLK, N_CLASS), 1)
    onehot = (cols == labs).astype(x.dtype)
    per = _aam_per_row(x, onehot)

    @pl.when(i == 0)
    def _():
        psum_ref[...] = jnp.zeros_like(psum_ref)

    psum_ref[...] += jnp.sum(per, axis=0, keepdims=True)


def _center_body(sums_ref, cntcol_ref, csum_ref, npres_ref):
    sums = sums_ref[...]
    cnt = cntcol_ref[...]
    present = cnt > 0
    inv = jnp.where(present, 1.0 / jnp.where(present, cnt, 1.0), 0.0)
    centers = sums * inv
    rows = jax.lax.broadcasted_iota(jnp.int32, (N_CLASS, N_CLASS), 0)
    cols = jax.lax.broadcasted_iota(jnp.int32, (N_CLASS, N_CLASS), 1)
    diag = (rows == cols).astype(sums.dtype)
    per = _aam_per_row(centers, diag)
    per = jnp.where(present, per, 0.0)
    csum_ref[...] = jnp.sum(per, axis=0, keepdims=True)
    npres_ref[...] = jnp.sum(present.astype(jnp.float32), axis=0, keepdims=True)


_CSLC = 32
_NFULL = 31
_CLAST = N_CLASS - _NFULL * _CSLC
_RCHUNK = 1024
_RCHUNK8 = 1024
_L = 16


def _sc_scatter_body(x_hbm, lab_hbm, sums_hbm, cnt_hbm, labs_v, stage_a,
                     stage_b, acc_v, stage8_v, acc8_v, cnt_v, sem_a, sem_b,
                     nrows=BATCH):
    c = lax.axis_index("c")
    s = lax.axis_index("s")
    wid = s * _NC + c
    c0 = wid * _CSLC

    def zrow(i, carry):
        acc_v[i, pl.ds(0, _L)] = jnp.zeros((_L,), jnp.float32)
        acc_v[i, pl.ds(_L, _L)] = jnp.zeros((_L,), jnp.float32)
        return carry

    lax.fori_loop(0, N_CLASS, zrow, 0, unroll=8)

    pltpu.sync_copy(lab_hbm, labs_v)

    iota = lax.iota(jnp.int32, _L)

    def make_row(k, stage_v):
        def row(g, carry):
            lr = g * _L
            lv = labs_v[pl.ds(k * _RCHUNK + lr, _L)]
            for j in range(_L):
                lab = lv[j]
                v0 = stage_v[lr + j, pl.ds(0, _L)]
                v1 = stage_v[lr + j, pl.ds(_L, _L)]
                plsc.addupdate(acc_v.at[lab, pl.ds(0, _L)], v0)
                plsc.addupdate(acc_v.at[lab, pl.ds(_L, _L)], v1)
            return carry

        return row

    @pl.when(wid < _NFULL)
    def _():
        nchunks = nrows // _RCHUNK
        bufs = [stage_a, stage_b]
        sems = [sem_a, sem_b]

        def start(k):
            return pltpu.async_copy(
                x_hbm.at[pl.ds(k * _RCHUNK, _RCHUNK), pl.ds(c0, _CSLC)],
                bufs[k % 2],
                sems[k % 2],
            )

        cp = start(0)
        for k in range(nchunks):
            cp.wait()
            if k + 1 < nchunks:
                cp = start(k + 1)
            lax.fori_loop(0, _RCHUNK // _L, make_row(k, bufs[k % 2]), 0,
                          unroll=4)
        pltpu.sync_copy(acc_v, sums_hbm.at[:, pl.ds(c0, _CSLC)])

    @pl.when(wid == _NFULL)
    def _():
        def zcnt(i, carry):
            cnt_v[pl.ds(i * _L, _L)] = jnp.zeros((_L,), jnp.float32)
            return carry

        lax.fori_loop(0, 63, zcnt, 0, unroll=8)
        ones = jnp.ones((_L,), jnp.float32)

        def zrow8(i, carry):
            acc8_v[i, pl.ds(0, _L)] = jnp.zeros((_L,), jnp.float32)
            return carry

        lax.fori_loop(0, N_CLASS, zrow8, 0, unroll=8)

        for k in range(nrows // _RCHUNK8):
            pltpu.sync_copy(
                x_hbm.at[pl.ds(k * _RCHUNK8, _RCHUNK8), pl.ds(c0, _CLAST)],
                stage8_v,
            )

            def grp8(g, carry):
                lr = g * _L
                labs16 = labs_v[pl.ds(k * _RCHUNK8 + lr, _L)]
                rows = lr + iota
                cvecs = [(iota + col) & (_CLAST - 1) for col in range(_CLAST)]
                vals = [
                    plsc.load_gather(stage8_v, [rows, cv]) for cv in cvecs
                ]
                for cv, v in zip(cvecs, vals):
                    plsc.addupdate_scatter(acc8_v, [labs16, cv], v)
                plsc.addupdate_scatter(cnt_v, [labs16], ones)
                return carry

            lax.fori_loop(0, _RCHUNK8 // _L, grp8, 0, unroll=2)
        pltpu.sync_copy(acc8_v.at[:, pl.ds(0, _CLAST)],
                        sums_hbm.at[:, pl.ds(c0, _CLAST)])
        pltpu.sync_copy(cnt_v.at[pl.ds(0, N_CLASS)], cnt_hbm)


_sc_scatter = functools.partial(
    pl.kernel,
    mesh=plsc.VectorSubcoreMesh(core_axis_name="c", subcore_axis_name="s"),
    compiler_params=pltpu.CompilerParams(
        use_tc_tiling_on_sc=False, needs_layout_passes=False
    ),
    out_type=[
        jax.ShapeDtypeStruct((N_CLASS, N_CLASS), jnp.float32),
        jax.ShapeDtypeStruct((N_CLASS,), jnp.float32),
    ],
    scratch_types=[
        pltpu.VMEM((BATCH,), jnp.int32),
        pltpu.VMEM((_RCHUNK, _CSLC), jnp.float32),
        pltpu.VMEM((_RCHUNK, _CSLC), jnp.float32),
        pltpu.VMEM((N_CLASS, _CSLC), jnp.float32),
        pltpu.VMEM((_RCHUNK8, _CLAST), jnp.float32),
        pltpu.VMEM((N_CLASS, _L), jnp.float32),
        pltpu.VMEM((1008,), jnp.float32),
        pltpu.SemaphoreType.DMA,
        pltpu.SemaphoreType.DMA,
    ],
)(_sc_scatter_body)


def kernel(x, label):
    sums, cnt = _sc_scatter(x, label)

    lab3 = label.reshape(NBLK, BLK, 1)
    psum = pl.pallas_call(
        _sample_body,
        grid=(NBLK,),
        in_specs=[
            pl.BlockSpec((BLK, N_CLASS), lambda i: (i, 0)),
            pl.BlockSpec((1, BLK, 1), lambda i: (i, 0, 0)),
        ],
        out_specs=pl.BlockSpec((1, 1), lambda i: (0, 0)),
        out_shape=jax.ShapeDtypeStruct((1, 1), jnp.float32),
    )(x, lab3)

    csum, npres = pl.pallas_call(
        _center_body,
        out_shape=[
            jax.ShapeDtypeStruct((1, 1), jnp.float32),
            jax.ShapeDtypeStruct((1, 1), jnp.float32),
        ],
    )(sums, cnt.reshape(N_CLASS, 1))

    loss = 0.5 * psum[0, 0] / BATCH + 0.5 * csum[0, 0] / npres[0, 0]
    return loss


# speedup vs baseline: 1.0468x; 1.0020x over previous
import functools
import math

import jax
import jax.numpy as jnp
from jax import lax
from jax.experimental import pallas as pl
from jax.experimental.pallas import tpu as pltpu
from jax.experimental.pallas import tpu_sc as plsc

N_CLASS = 1000
BATCH = 4096
M = 0.2
S = 30.0
EPS = 0.1

_COS_M = math.cos(M)
_SIN_M = math.sin(M)
_TH = math.cos(M)
_MM = math.sin(math.pi - M) * M

BLK = 512
NBLK = BATCH // BLK

_NC = 2
_NS = 16
_NW = _NC * _NS


def _aam_per_row(vals, onehot):
    c_lab = jnp.sum(vals * onehot, axis=1, keepdims=True)
    sine = jnp.sqrt(jnp.clip(1.0 - c_lab * c_lab, 0.0, 1.0))
    phi = c_lab * _COS_M - sine * _SIN_M
    phi = jnp.where(c_lab - _TH > 0, phi, c_lab - _MM)
    delta = S * (phi - c_lab)
    t = S * vals
    t_mod = t + onehot * delta
    t_sum = jnp.sum(t, axis=1, keepdims=True)
    m = jnp.max(t_mod, axis=1, keepdims=True)
    sumexp = jnp.sum(jnp.exp(t_mod - m), axis=1, keepdims=True)
    lse = m + jnp.log(sumexp)
    rmean = (t_sum + delta) / N_CLASS
    nll = lse - S * phi
    smooth = lse - rmean
    return (1.0 - EPS) * nll + EPS * smooth


def _sample_body(x_ref, lab_ref, psum_ref):
    i = pl.program_id(0)
    x = x_ref[...]
    labs = lab_ref[0]
    cols = jax.lax.broadcasted_iota(jnp.int32, (BLK, N_CLASS), 1)
    onehot = (cols == labs).astype(x.dtype)
    per = _aam_per_row(x, onehot)

    @pl.when(i == 0)
    def _():
        psum_ref[...] = jnp.zeros_like(psum_ref)

    psum_ref[...] += jnp.sum(per, axis=0, keepdims=True)


def _center_body(sums_ref, cntcol_ref, csum_ref, npres_ref):
    sums = sums_ref[...]
    cnt = cntcol_ref[...]
    present = cnt > 0
    inv = jnp.where(present, 1.0 / jnp.where(present, cnt, 1.0), 0.0)
    centers = sums * inv
    rows = jax.lax.broadcasted_iota(jnp.int32, (N_CLASS, N_CLASS), 0)
    cols = jax.lax.broadcasted_iota(jnp.int32, (N_CLASS, N_CLASS), 1)
    diag = (rows == cols).astype(sums.dtype)
    per = _aam_per_row(centers, diag)
    per = jnp.where(present, per, 0.0)
    csum_ref[...] = jnp.sum(per, axis=0, keepdims=True)
    npres_ref[...] = jnp.sum(present.astype(jnp.float32), axis=0, keepdims=True)


_CSLC = 32
_NFULL = 31
_CLAST = N_CLASS - _NFULL * _CSLC
_RCHUNK = 1024
_RCHUNK8 = 512
_L = 16


def _sc_scatter_body(x_hbm, lab_hbm, sums_hbm, cnt_hbm, labs_v, stage_a,
                     stage_b, acc_v, stage8_a, stage8_b, acc8_v, cnt_v,
                     sem_a, sem_b, nrows=BATCH):
    c = lax.axis_index("c")
    s = lax.axis_index("s")
    wid = s * _NC + c
    c0 = wid * _CSLC

    def zrow(i, carry):
        acc_v[i, pl.ds(0, _L)] = jnp.zeros((_L,), jnp.float32)
        acc_v[i, pl.ds(_L, _L)] = jnp.zeros((_L,), jnp.float32)
        return carry

    lax.fori_loop(0, N_CLASS, zrow, 0, unroll=8)

    pltpu.sync_copy(lab_hbm, labs_v)

    iota = lax.iota(jnp.int32, _L)

    def make_row(k, stage_v):
        def row(g, carry):
            lr = g * _L
            lv = labs_v[pl.ds(k * _RCHUNK + lr, _L)]
            for j in range(_L):
                lab = lv[j]
                v0 = stage_v[lr + j, pl.ds(0, _L)]
                v1 = stage_v[lr + j, pl.ds(_L, _L)]
                plsc.addupdate(acc_v.at[lab, pl.ds(0, _L)], v0)
                plsc.addupdate(acc_v.at[lab, pl.ds(_L, _L)], v1)
            return carry

        return row

    @pl.when(wid < _NFULL)
    def _():
        nchunks = nrows // _RCHUNK
        bufs = [stage_a, stage_b]
        sems = [sem_a, sem_b]

        def start(k):
            return pltpu.async_copy(
                x_hbm.at[pl.ds(k * _RCHUNK, _RCHUNK), pl.ds(c0, _CSLC)],
                bufs[k % 2],
                sems[k % 2],
            )

        cp = start(0)
        for k in range(nchunks):
            cp.wait()
            if k + 1 < nchunks:
                cp = start(k + 1)
            lax.fori_loop(0, _RCHUNK // _L, make_row(k, bufs[k % 2]), 0,
                          unroll=4)
        pltpu.sync_copy(acc_v, sums_hbm.at[:, pl.ds(c0, _CSLC)])

    @pl.when(wid == _NFULL)
    def _():
        def zcnt(i, carry):
            cnt_v[pl.ds(i * _L, _L)] = jnp.zeros((_L,), jnp.float32)
            return carry

        lax.fori_loop(0, 63, zcnt, 0, unroll=8)
        ones = jnp.ones((_L,), jnp.float32)

        def zrow8(i, carry):
            acc8_v[i, pl.ds(0, _L)] = jnp.zeros((_L,), jnp.float32)
            return carry

        lax.fori_loop(0, N_CLASS, zrow8, 0, unroll=8)

        bufs8 = [stage8_a, stage8_b]
        sems8 = [sem_a, sem_b]
        nchunks8 = nrows // _RCHUNK8

        def start8(k):
            return pltpu.async_copy(
                x_hbm.at[pl.ds(k * _RCHUNK8, _RCHUNK8), pl.ds(c0, _CLAST)],
                bufs8[k % 2],
                sems8[k % 2],
            )

        def make_grp8(k, stage8_v):
            def grp8(g, carry):
                lr = g * _L
                labs16 = labs_v[pl.ds(k * _RCHUNK8 + lr, _L)]
                rows = lr + iota
                cvecs = [(iota + col) & (_CLAST - 1) for col in range(_CLAST)]
                vals = [
                    plsc.load_gather(stage8_v, [rows, cv]) for cv in cvecs
                ]
                for cv, v in zip(cvecs, vals):
                    plsc.addupdate_scatter(acc8_v, [labs16, cv], v)
                plsc.addupdate_scatter(cnt_v, [labs16], ones)
                return carry

            return grp8

        cp8 = start8(0)
        for k in range(nchunks8):
            cp8.wait()
            if k + 1 < nchunks8:
                cp8 = start8(k + 1)
            lax.fori_loop(0, _RCHUNK8 // _L, make_grp8(k, bufs8[k % 2]), 0,
                          unroll=2)
        pltpu.sync_copy(acc8_v.at[:, pl.ds(0, _CLAST)],
                        sums_hbm.at[:, pl.ds(c0, _CLAST)])
        pltpu.sync_copy(cnt_v.at[pl.ds(0, N_CLASS)], cnt_hbm)


_sc_scatter = functools.partial(
    pl.kernel,
    mesh=plsc.VectorSubcoreMesh(core_axis_name="c", subcore_axis_name="s"),
    compiler_params=pltpu.CompilerParams(
        use_tc_tiling_on_sc=False, needs_layout_passes=False
    ),
    out_type=[
        jax.ShapeDtypeStruct((N_CLASS, N_CLASS), jnp.float32),
        jax.ShapeDtypeStruct((N_CLASS,), jnp.float32),
    ],
    scratch_types=[
        pltpu.VMEM((BATCH,), jnp.int32),
        pltpu.VMEM((_RCHUNK, _CSLC), jnp.float32),
        pltpu.VMEM((_RCHUNK, _CSLC), jnp.float32),
        pltpu.VMEM((N_CLASS, _CSLC), jnp.float32),
        pltpu.VMEM((_RCHUNK8, _CLAST), jnp.float32),
        pltpu.VMEM((_RCHUNK8, _CLAST), jnp.float32),
        pltpu.VMEM((N_CLASS, _L), jnp.float32),
        pltpu.VMEM((1008,), jnp.float32),
        pltpu.SemaphoreType.DMA,
        pltpu.SemaphoreType.DMA,
    ],
)(_sc_scatter_body)


def kernel(x, label):
    sums, cnt = _sc_scatter(x, label)

    lab3 = label.reshape(NBLK, BLK, 1)
    psum = pl.pallas_call(
        _sample_body,
        grid=(NBLK,),
        in_specs=[
            pl.BlockSpec((BLK, N_CLASS), lambda i: (i, 0)),
            pl.BlockSpec((1, BLK, 1), lambda i: (i, 0, 0)),
        ],
        out_specs=pl.BlockSpec((1, 1), lambda i: (0, 0)),
        out_shape=jax.ShapeDtypeStruct((1, 1), jnp.float32),
    )(x, lab3)

    csum, npres = pl.pallas_call(
        _center_body,
        out_shape=[
            jax.ShapeDtypeStruct((1, 1), jnp.float32),
            jax.ShapeDtypeStruct((1, 1), jnp.float32),
        ],
    )(sums, cnt.reshape(N_CLASS, 1))

    loss = 0.5 * psum[0, 0] / BATCH + 0.5 * csum[0, 0] / npres[0, 0]
    return loss
